# baseline (speedup 1.0000x reference)
import functools

import jax
import jax.numpy as jnp
from jax import lax
from jax.experimental import pallas as pl
from jax.experimental.pallas import tpu as pltpu
from jax.experimental.pallas import tpu_sc as plsc

_B, _C, _H, _W, _N = 4, 768, 224, 224, 4096
_HW = _H * _W
_BC = _B * _C
_NW = 32
_L = 16
_PAIRS = _BC // _NW
_NCHUNK = _N // _L


def _sc_body(o1_hbm, o2_hbm, r1_hbm, c1_hbm, r2_hbm, c2_hbm, out_hbm,
             p1_v, p2_v, r1_v, c1_v, r2_v, c2_v, acc_v):
    cid = lax.axis_index("c")
    sid = lax.axis_index("s")
    wid = sid * 2 + cid
    b = wid // 8
    p0 = wid * _PAIRS

    pltpu.sync_copy(r1_hbm.at[b], r1_v)
    pltpu.sync_copy(c1_hbm.at[b], c1_v)
    pltpu.sync_copy(r2_hbm.at[b], r2_v)
    pltpu.sync_copy(c2_hbm.at[b], c2_v)

    acc_v[...] = jnp.zeros((_L,), jnp.float32)

    def chan_body(j, carry):
        p = p0 + j
        pltpu.sync_copy(o1_hbm.at[p], p1_v)
        pltpu.sync_copy(o2_hbm.at[p], p2_v)

        def inner(k, c2_):
            s = k * _L
            idx1 = r1_v[pl.ds(s, _L)] * _W + c1_v[pl.ds(s, _L)]
            idx2 = r2_v[pl.ds(s, _L)] * _W + c2_v[pl.ds(s, _L)]
            v1 = plsc.load_gather(p1_v, [idx1])
            v2 = plsc.load_gather(p2_v, [idx2])
            d = v1 - v2
            acc_v[...] = acc_v[...] + d * d
            return c2_

        lax.fori_loop(0, _NCHUNK, inner, 0)
        return carry

    lax.fori_loop(0, _PAIRS, chan_body, 0)
    pltpu.sync_copy(acc_v, out_hbm.at[wid])


@jax.jit
def _sc_loss(o1, o2, r1, c1, r2, c2):
    mesh = plsc.VectorSubcoreMesh(core_axis_name="c", subcore_axis_name="s")
    parts = pl.kernel(
        _sc_body,
        out_type=jax.ShapeDtypeStruct((_NW, _L), jnp.float32),
        mesh=mesh,
        compiler_params=pltpu.CompilerParams(needs_layout_passes=False),
        scratch_types=[
            pltpu.VMEM((_HW,), jnp.float32),
            pltpu.VMEM((_HW,), jnp.float32),
            pltpu.VMEM((_N,), jnp.int32),
            pltpu.VMEM((_N,), jnp.int32),
            pltpu.VMEM((_N,), jnp.int32),
            pltpu.VMEM((_N,), jnp.int32),
            pltpu.VMEM((_L,), jnp.float32),
        ],
    )(o1, o2, r1, c1, r2, c2)
    return jnp.sum(parts) * (1.0 / (_B * _N))


def kernel(out_1, out_2, match_1, match_2, nonmatch_2):
    del nonmatch_2
    o1 = out_1.reshape(_BC, _HW)
    o2 = out_2.reshape(_BC, _HW)
    r1 = match_1[:, :, 0]
    c1 = match_1[:, :, 1]
    r2 = match_2[:, :, 0]
    c2 = match_2[:, :, 1]
    return _sc_loss(o1, o2, r1, c1, r2, c2)


# speedup vs baseline: 1.4838x; 1.4838x over previous
import functools

import jax
import jax.numpy as jnp
from jax import lax
from jax.experimental import pallas as pl
from jax.experimental.pallas import tpu as pltpu
from jax.experimental.pallas import tpu_sc as plsc

_B, _C, _H, _W, _N = 4, 768, 224, 224, 4096
_NW = 32
_L = 16
_PAIRS = _B * _C // _NW
_NCHUNK = _N // _L


def _sc_body(o1_hbm, o2_hbm, m1_hbm, m2_hbm, out_hbm,
             plane_a, plane_b, vals_v, m1_v, m2_v, acc_v,
             sem_a, sem_b):
    cid = lax.axis_index("c")
    sid = lax.axis_index("s")
    wid = sid * 2 + cid
    b = wid // 8
    ch0 = (wid % 8) * _PAIRS

    pltpu.sync_copy(m1_hbm.at[b], m1_v)
    pltpu.sync_copy(m2_hbm.at[b], m2_v)

    acc_v[...] = jnp.zeros((_L,), jnp.float32)

    def start_a(ch):
        pltpu.async_copy(o1_hbm.at[b, ch], plane_a, sem_a)

    def start_b(ch):
        pltpu.async_copy(o2_hbm.at[b, ch], plane_b, sem_b)

    def wait_a(ch):
        pltpu.make_async_copy(o1_hbm.at[b, ch], plane_a, sem_a).wait()

    def wait_b(ch):
        pltpu.make_async_copy(o2_hbm.at[b, ch], plane_b, sem_b).wait()

    def gather1(_):
        def body(k, u):
            s = k * _L
            m = m1_v[pl.ds(s, _L)]
            vals_v[pl.ds(s, _L)] = plsc.load_gather(
                plane_a, [m >> 16, m & 0xFFFF])
            return u

        lax.fori_loop(0, _NCHUNK, body, 0, unroll=4)

    def gather2_acc(_):
        def body(k, u):
            s = k * _L
            m = m2_v[pl.ds(s, _L)]
            v2 = plsc.load_gather(plane_b, [m >> 16, m & 0xFFFF])
            d = vals_v[pl.ds(s, _L)] - v2
            acc_v[...] = acc_v[...] + d * d
            return u

        lax.fori_loop(0, _NCHUNK, body, 0, unroll=4)

    start_a(ch0)
    start_b(ch0)

    def plane_body(j, u):
        ch = ch0 + j
        wait_a(ch)
        gather1(None)
        start_a(ch + 1)
        wait_b(ch)
        gather2_acc(None)
        start_b(ch + 1)
        return u

    lax.fori_loop(0, _PAIRS - 1, plane_body, 0)
    ch_last = ch0 + _PAIRS - 1
    wait_a(ch_last)
    gather1(None)
    wait_b(ch_last)
    gather2_acc(None)

    pltpu.sync_copy(acc_v, out_hbm.at[wid])


@jax.jit
def _sc_loss(o1, o2, m1, m2):
    mesh = plsc.VectorSubcoreMesh(core_axis_name="c", subcore_axis_name="s")
    parts = pl.kernel(
        _sc_body,
        out_type=jax.ShapeDtypeStruct((_NW, _L), jnp.float32),
        mesh=mesh,
        compiler_params=pltpu.CompilerParams(needs_layout_passes=False),
        scratch_types=[
            pltpu.VMEM((_H, _W), jnp.float32),
            pltpu.VMEM((_H, _W), jnp.float32),
            pltpu.VMEM((_N,), jnp.float32),
            pltpu.VMEM((_N,), jnp.int32),
            pltpu.VMEM((_N,), jnp.int32),
            pltpu.VMEM((_L,), jnp.float32),
            pltpu.SemaphoreType.DMA,
            pltpu.SemaphoreType.DMA,
        ],
    )(o1, o2, m1, m2)
    return jnp.sum(parts) * (1.0 / (_B * _N))


def kernel(out_1, out_2, match_1, match_2, nonmatch_2):
    del nonmatch_2
    m1 = (match_1[:, :, 0] << 16) | match_1[:, :, 1]
    m2 = (match_2[:, :, 0] << 16) | match_2[:, :, 1]
    return _sc_loss(out_1, out_2, m1, m2)
